# baseline (speedup 1.0000x reference)
from functools import partial

import jax
import jax.numpy as jnp
from jax.experimental import pallas as pl
from jax.experimental.pallas import tpu as pltpu

TOPK = 5
NC = 8
NEG = float("-inf")
POS = float("inf")


def _block_topk_body(qt_ref, k_ref, v_out, i_out, *, bk, k_total, q):
    kb = pl.program_id(0)
    sims = jnp.dot(k_ref[...], qt_ref[...],
                   preferred_element_type=jnp.float32)
    iota_f = jax.lax.broadcasted_iota(jnp.int32, (bk, q), 0).astype(jnp.float32)
    gidx = (kb * bk).astype(jnp.float32) + iota_f
    sims = jnp.where(gidx < k_total, sims, NEG)
    vrows, irows = [], []
    for _ in range(TOPK):
        m = jnp.max(sims, axis=0, keepdims=True)
        c = jnp.min(jnp.where(sims == m, gidx, POS), axis=0, keepdims=True)
        sims = jnp.where(gidx == c, NEG, sims)
        vrows.append(m)
        irows.append(c)
    for _ in range(NC - TOPK):
        vrows.append(jnp.full((1, q), NEG, jnp.float32))
        irows.append(jnp.full((1, q), POS, jnp.float32))
    v_out[0, :, :] = jnp.concatenate(vrows, axis=0)
    i_out[0, :, :] = jnp.concatenate(irows, axis=0)


def _merge_topk_body(cv_ref, ci_ref, tv_ref, ti_ref, *, q):
    v = cv_ref[...]
    idx = ci_ref[...]
    vrows, irows = [], []
    for _ in range(TOPK):
        m = jnp.max(v, axis=0, keepdims=True)
        c = jnp.min(jnp.where(v == m, idx, POS), axis=0, keepdims=True)
        v = jnp.where(idx == c, NEG, v)
        vrows.append(m)
        irows.append(c)
    for _ in range(NC - TOPK):
        vrows.append(jnp.full((1, q), NEG, jnp.float32))
        irows.append(jnp.full((1, q), POS, jnp.float32))
    tv_ref[...] = jnp.concatenate(vrows, axis=0)
    ti_ref[...] = jnp.concatenate(irows, axis=0)


def kernel(queries, keys):
    q, d = queries.shape
    k_total = keys.shape[0]
    bk = 2048
    nkb = -(-k_total // bk)
    qt = queries.T

    cand_v, cand_i = pl.pallas_call(
        partial(_block_topk_body, bk=bk, k_total=k_total, q=q),
        grid=(nkb,),
        in_specs=[
            pl.BlockSpec((d, q), lambda i: (0, 0)),
            pl.BlockSpec((bk, d), lambda i: (i, 0)),
        ],
        out_specs=[
            pl.BlockSpec((1, NC, q), lambda i: (i, 0, 0)),
            pl.BlockSpec((1, NC, q), lambda i: (i, 0, 0)),
        ],
        out_shape=[
            jax.ShapeDtypeStruct((nkb, NC, q), jnp.float32),
            jax.ShapeDtypeStruct((nkb, NC, q), jnp.float32),
        ],
        compiler_params=pltpu.CompilerParams(
            dimension_semantics=("arbitrary",)),
    )(qt, keys)

    height = nkb * NC
    cv = cand_v.reshape(height, q)
    ci = cand_i.reshape(height, q)

    tv, ti = pl.pallas_call(
        partial(_merge_topk_body, q=q),
        in_specs=[
            pl.BlockSpec((height, q), lambda: (0, 0)),
            pl.BlockSpec((height, q), lambda: (0, 0)),
        ],
        out_specs=[
            pl.BlockSpec((NC, q), lambda: (0, 0)),
            pl.BlockSpec((NC, q), lambda: (0, 0)),
        ],
        out_shape=[
            jax.ShapeDtypeStruct((NC, q), jnp.float32),
            jax.ShapeDtypeStruct((NC, q), jnp.float32),
        ],
    )(cv, ci)
    return tv[:TOPK, :].T, ti[:TOPK, :].T.astype(jnp.int32)


# speedup vs baseline: 1.0643x; 1.0643x over previous
from functools import partial

import jax
import jax.numpy as jnp
from jax.experimental import pallas as pl
from jax.experimental.pallas import tpu as pltpu

TOPK = 5
NC = 8
NEG = float("-inf")
POS = float("inf")


def _block_topk_body(q_ref, k_ref, v_out, i_out, *, bk, k_total, q):
    kb = pl.program_id(0)
    sims = jnp.dot(q_ref[...], k_ref[...].T,
                   preferred_element_type=jnp.float32)
    lidx = jax.lax.broadcasted_iota(jnp.int32, (q, bk), 1).astype(jnp.float32)
    nvalid = (k_total - kb * bk).astype(jnp.float32)
    sims = jnp.where(lidx < nvalid, sims, NEG)
    base = (kb * bk).astype(jnp.float32)
    vcols, icols = [], []
    for _ in range(TOPK):
        m = jnp.max(sims, axis=1, keepdims=True)
        c = jnp.min(jnp.where(sims == m, lidx, POS), axis=1, keepdims=True)
        sims = jnp.where(lidx == c, NEG, sims)
        vcols.append(m)
        icols.append(c + base)
    for _ in range(NC - TOPK):
        vcols.append(jnp.full((q, 1), NEG, jnp.float32))
        icols.append(jnp.full((q, 1), POS, jnp.float32))
    v_out[0, :, :] = jnp.concatenate(vcols, axis=1)
    i_out[0, :, :] = jnp.concatenate(icols, axis=1)


def _merge_topk_body(cv_ref, ci_ref, tv_ref, ti_ref, *, q):
    v = cv_ref[...]
    idx = ci_ref[...]
    vcols, icols = [], []
    for _ in range(TOPK):
        m = jnp.max(v, axis=1, keepdims=True)
        c = jnp.min(jnp.where(v == m, idx, POS), axis=1, keepdims=True)
        v = jnp.where(idx == c, NEG, v)
        vcols.append(m)
        icols.append(c)
    for _ in range(NC - TOPK):
        vcols.append(jnp.full((q, 1), NEG, jnp.float32))
        icols.append(jnp.full((q, 1), POS, jnp.float32))
    tv_ref[...] = jnp.concatenate(vcols, axis=1)
    ti_ref[...] = jnp.concatenate(icols, axis=1).astype(jnp.int32)


def kernel(queries, keys):
    q, d = queries.shape
    k_total = keys.shape[0]
    bk = 2048
    nkb = -(-k_total // bk)

    cand_v, cand_i = pl.pallas_call(
        partial(_block_topk_body, bk=bk, k_total=k_total, q=q),
        grid=(nkb,),
        in_specs=[
            pl.BlockSpec((q, d), lambda i: (0, 0)),
            pl.BlockSpec((bk, d), lambda i: (i, 0)),
        ],
        out_specs=[
            pl.BlockSpec((1, q, NC), lambda i: (i, 0, 0)),
            pl.BlockSpec((1, q, NC), lambda i: (i, 0, 0)),
        ],
        out_shape=[
            jax.ShapeDtypeStruct((nkb, q, NC), jnp.float32),
            jax.ShapeDtypeStruct((nkb, q, NC), jnp.float32),
        ],
        compiler_params=pltpu.CompilerParams(
            dimension_semantics=("arbitrary",)),
    )(queries, keys)

    width = nkb * NC
    cv = cand_v.transpose(1, 0, 2).reshape(q, width)
    ci = cand_i.transpose(1, 0, 2).reshape(q, width)

    tv, ti = pl.pallas_call(
        partial(_merge_topk_body, q=q),
        in_specs=[
            pl.BlockSpec((q, width), lambda: (0, 0)),
            pl.BlockSpec((q, width), lambda: (0, 0)),
        ],
        out_specs=[
            pl.BlockSpec((q, NC), lambda: (0, 0)),
            pl.BlockSpec((q, NC), lambda: (0, 0)),
        ],
        out_shape=[
            jax.ShapeDtypeStruct((q, NC), jnp.float32),
            jax.ShapeDtypeStruct((q, NC), jnp.int32),
        ],
    )(cv, ci)
    return tv[:, :TOPK], ti[:, :TOPK]


# speedup vs baseline: 1.0656x; 1.0012x over previous
from functools import partial

import jax
import jax.numpy as jnp
from jax.experimental import pallas as pl
from jax.experimental.pallas import tpu as pltpu

TOPK = 5
NC = 8
NEG = float("-inf")
POS = float("inf")


def _stream_body(q_ref, k_ref, tv_ref, ti_ref, sims_scr, si_scr,
                 *, bk, k_total, q, nkb):
    i = pl.program_id(0)

    @pl.when(i == 0)
    def _init():
        tv_ref[...] = jnp.full((q, NC), NEG, jnp.float32)
        si_scr[...] = jnp.full((q, NC), POS, jnp.float32)

    @pl.when(i < nkb)
    def _dot():
        sims_scr[i % 2] = jnp.dot(q_ref[...], k_ref[...].T,
                                  preferred_element_type=jnp.float32)

    @pl.when(i > 0)
    def _extract():
        b = i - 1
        buf = (i + 1) % 2
        lidx = jax.lax.broadcasted_iota(
            jnp.int32, (q, bk), 1).astype(jnp.float32)

        @pl.when(b == nkb - 1)
        def _mask_tail():
            nvalid = (k_total - b * bk).astype(jnp.float32)
            sims_scr[buf] = jnp.where(
                lidx < nvalid, sims_scr[buf], NEG)

        base = (b * bk).astype(jnp.float32)
        lane = jax.lax.broadcasted_iota(
            jnp.int32, (q, NC), 1).astype(jnp.float32)
        t = tv_ref[:, TOPK - 1:TOPK]
        sims0 = sims_scr[buf]
        g = jnp.max(jnp.sum((sims0 >= t).astype(jnp.float32),
                            axis=1, keepdims=True))

        for j in range(TOPK):
            @pl.when(g > j)
            def _iter():
                sims = sims_scr[buf]
                m = jnp.max(sims, axis=1, keepdims=True)
                c = jnp.min(jnp.where(sims == m, lidx, POS),
                            axis=1, keepdims=True)
                sims_scr[buf] = jnp.where(lidx == c, NEG, sims)
                cg = c + base
                sv = tv_ref[...]
                si = si_scr[...]
                before = (sv > m) | ((sv == m) & (si < cg))
                pos = jnp.sum(before.astype(jnp.float32),
                              axis=1, keepdims=True)
                sh_v = jnp.concatenate([sv[:, :1], sv[:, :-1]], axis=1)
                sh_i = jnp.concatenate([si[:, :1], si[:, :-1]], axis=1)
                tv_ref[...] = jnp.where(
                    lane < pos, sv, jnp.where(lane == pos, m, sh_v))
                si_scr[...] = jnp.where(
                    lane < pos, si, jnp.where(lane == pos, cg, sh_i))

    @pl.when(i == nkb)
    def _finalize():
        ti_ref[...] = si_scr[...].astype(jnp.int32)


def kernel(queries, keys):
    q, d = queries.shape
    k_total = keys.shape[0]
    bk = 2048
    nkb = -(-k_total // bk)

    tv, ti = pl.pallas_call(
        partial(_stream_body, bk=bk, k_total=k_total, q=q, nkb=nkb),
        grid=(nkb + 1,),
        in_specs=[
            pl.BlockSpec((q, d), lambda i: (0, 0)),
            pl.BlockSpec((bk, d), lambda i: (jnp.minimum(i, nkb - 1), 0)),
        ],
        out_specs=[
            pl.BlockSpec((q, NC), lambda i: (0, 0)),
            pl.BlockSpec((q, NC), lambda i: (0, 0)),
        ],
        out_shape=[
            jax.ShapeDtypeStruct((q, NC), jnp.float32),
            jax.ShapeDtypeStruct((q, NC), jnp.int32),
        ],
        scratch_shapes=[
            pltpu.VMEM((2, q, bk), jnp.float32),
            pltpu.VMEM((q, NC), jnp.float32),
        ],
        compiler_params=pltpu.CompilerParams(
            dimension_semantics=("arbitrary",)),
    )(queries, keys)
    return tv[:, :TOPK], ti[:, :TOPK]


# speedup vs baseline: 1.1143x; 1.0457x over previous
from functools import partial

import jax
import jax.numpy as jnp
from jax.experimental import pallas as pl
from jax.experimental.pallas import tpu as pltpu

TOPK = 5
NC = 8
NEG = float("-inf")
POS = float("inf")


def _block_topk_body(q_ref, k_ref, v_out, i_out, *, bk, k_total, q):
    kb = pl.program_id(0)
    sims = jnp.dot(q_ref[...], k_ref[...].T,
                   preferred_element_type=jnp.float32)
    lidx = jax.lax.broadcasted_iota(jnp.int32, (q, bk), 1).astype(jnp.float32)
    nvalid = (k_total - kb * bk).astype(jnp.float32)
    sims = jnp.where(lidx < nvalid, sims, NEG)
    base = (kb * bk).astype(jnp.float32)
    vcols, icols = [], []
    for _ in range(TOPK):
        m = jnp.max(sims, axis=1, keepdims=True)
        c = jnp.min(jnp.where(sims == m, lidx, POS), axis=1, keepdims=True)
        sims = jnp.where(lidx == c, NEG, sims)
        vcols.append(m)
        icols.append(c + base)
    for _ in range(NC - TOPK):
        vcols.append(jnp.full((q, 1), NEG, jnp.float32))
        icols.append(jnp.full((q, 1), POS, jnp.float32))
    v_out[0, :, :] = jnp.concatenate(vcols, axis=1)
    i_out[0, :, :] = jnp.concatenate(icols, axis=1)


def _merge_topk_body(cv_ref, ci_ref, tv_ref, ti_ref, *, q):
    v = cv_ref[...]
    idx = ci_ref[...]
    vcols, icols = [], []
    for _ in range(TOPK):
        m = jnp.max(v, axis=1, keepdims=True)
        c = jnp.min(jnp.where(v == m, idx, POS), axis=1, keepdims=True)
        v = jnp.where(idx == c, NEG, v)
        vcols.append(m)
        icols.append(c)
    for _ in range(NC - TOPK):
        vcols.append(jnp.full((q, 1), NEG, jnp.float32))
        icols.append(jnp.full((q, 1), POS, jnp.float32))
    tv_ref[...] = jnp.concatenate(vcols, axis=1)
    ti_ref[...] = jnp.concatenate(icols, axis=1).astype(jnp.int32)


def kernel(queries, keys):
    q, d = queries.shape
    k_total = keys.shape[0]
    bk = 4096
    nkb = -(-k_total // bk)

    cand_v, cand_i = pl.pallas_call(
        partial(_block_topk_body, bk=bk, k_total=k_total, q=q),
        grid=(nkb,),
        in_specs=[
            pl.BlockSpec((q, d), lambda i: (0, 0)),
            pl.BlockSpec((bk, d), lambda i: (i, 0)),
        ],
        out_specs=[
            pl.BlockSpec((1, q, NC), lambda i: (i, 0, 0)),
            pl.BlockSpec((1, q, NC), lambda i: (i, 0, 0)),
        ],
        out_shape=[
            jax.ShapeDtypeStruct((nkb, q, NC), jnp.float32),
            jax.ShapeDtypeStruct((nkb, q, NC), jnp.float32),
        ],
        compiler_params=pltpu.CompilerParams(
            dimension_semantics=("arbitrary",)),
    )(queries, keys)

    width = nkb * NC
    cv = cand_v.transpose(1, 0, 2).reshape(q, width)
    ci = cand_i.transpose(1, 0, 2).reshape(q, width)

    tv, ti = pl.pallas_call(
        partial(_merge_topk_body, q=q),
        in_specs=[
            pl.BlockSpec((q, width), lambda: (0, 0)),
            pl.BlockSpec((q, width), lambda: (0, 0)),
        ],
        out_specs=[
            pl.BlockSpec((q, NC), lambda: (0, 0)),
            pl.BlockSpec((q, NC), lambda: (0, 0)),
        ],
        out_shape=[
            jax.ShapeDtypeStruct((q, NC), jnp.float32),
            jax.ShapeDtypeStruct((q, NC), jnp.int32),
        ],
    )(cv, ci)
    return tv[:, :TOPK], ti[:, :TOPK]
